# baseline (speedup 1.0000x reference)
import functools

import jax
import jax.numpy as jnp
from jax import lax
from jax.experimental import pallas as pl
from jax.experimental.pallas import tpu as pltpu
from jax.experimental.pallas import tpu_sc as plsc

VOCAB = 100000
DIM = 300
DIM_PAD = 304
B_TOTAL = 1024 * 200

_INFO = plsc.get_sparse_core_info()
_NC = _INFO.num_cores
_NS = _INFO.num_subcores
_NW = _NC * _NS
_B_PER_W = B_TOTAL // _NW
_CHUNK = 128
_NCHUNK = _B_PER_W // _CHUNK


def _sc_gather(idx2d, table_p):
    mesh = plsc.VectorSubcoreMesh(core_axis_name="c", subcore_axis_name="s")

    @functools.partial(
        pl.kernel,
        mesh=mesh,
        compiler_params=pltpu.CompilerParams(use_tc_tiling_on_sc=False),
        out_type=jax.ShapeDtypeStruct((B_TOTAL, DIM_PAD), jnp.float32),
        scratch_types=[
            pltpu.VMEM((_NCHUNK, _CHUNK), jnp.int32),
            pltpu.VMEM((_CHUNK, DIM_PAD), jnp.float32),
            pltpu.SemaphoreType.DMA,
        ],
    )
    def k(idx_hbm, table_hbm, out_hbm, idx_v, rows_v, sem):
        wid = lax.axis_index("s") * _NC + lax.axis_index("c")
        base = wid * _B_PER_W
        pltpu.sync_copy(idx_hbm.at[pl.ds(wid * _NCHUNK, _NCHUNK)], idx_v)

        def body(c, carry):
            off = pl.multiple_of(c * _CHUNK, 8)
            pltpu.async_copy(
                table_hbm.at[idx_v.at[c]], rows_v, sem
            ).wait()
            pltpu.sync_copy(rows_v, out_hbm.at[pl.ds(base + off, _CHUNK)])
            return carry

        lax.fori_loop(0, _NCHUNK, body, 0)

    return k(idx2d, table_p)


def kernel(x, table):
    idx2d = jnp.reshape(x, (_NW * _NCHUNK, _CHUNK)).astype(jnp.int32)
    table_p = jnp.pad(table, ((0, 0), (0, DIM_PAD - DIM)))
    out_p = _sc_gather(idx2d, table_p)
    return jnp.reshape(out_p[:, :DIM], (x.shape[0], x.shape[1], DIM))


# speedup vs baseline: 1.0092x; 1.0092x over previous
import functools

import jax
import jax.numpy as jnp
from jax import lax
from jax.experimental import pallas as pl
from jax.experimental.pallas import tpu as pltpu
from jax.experimental.pallas import tpu_sc as plsc

VOCAB = 100000
DIM = 300
DIM_PAD = 384
MR = DIM_PAD // 128
B_TOTAL = 1024 * 200

_INFO = plsc.get_sparse_core_info()
_NC = _INFO.num_cores
_NS = _INFO.num_subcores
_NW = _NC * _NS
_B_PER_W = B_TOTAL // _NW
_CHUNK = 128
_NCHUNK = _B_PER_W // _CHUNK
_IDXROWS_W = _B_PER_W * MR // 128


def _sc_gather(idx3, table3):
    mesh = plsc.VectorSubcoreMesh(core_axis_name="c", subcore_axis_name="s")

    @functools.partial(
        pl.kernel,
        mesh=mesh,
        compiler_params=pltpu.CompilerParams(use_tc_tiling_on_sc=False),
        out_type=jax.ShapeDtypeStruct((B_TOTAL * MR, 128), jnp.float32),
        scratch_types=[
            pltpu.VMEM((_IDXROWS_W, 128), jnp.int32),
            pltpu.VMEM((_CHUNK * MR, 128), jnp.float32),
            pltpu.SemaphoreType.DMA,
        ],
    )
    def k(idx_hbm, table_hbm, out_hbm, idx_v, rows_v, sem):
        wid = lax.axis_index("s") * _NC + lax.axis_index("c")
        base = wid * _B_PER_W * MR
        pltpu.sync_copy(idx_hbm.at[pl.ds(wid * _IDXROWS_W, _IDXROWS_W)], idx_v)

        def body(c, carry):
            off = pl.multiple_of(c * _CHUNK * MR, 8)
            copies = [
                pltpu.async_copy(
                    table_hbm.at[idx_v.at[c * MR + j]],
                    rows_v.at[pl.ds(j * 128, 128)],
                    sem,
                )
                for j in range(MR)
            ]
            for cp in copies:
                cp.wait()
            pltpu.sync_copy(rows_v, out_hbm.at[pl.ds(base + off, _CHUNK * MR)])
            return carry

        lax.fori_loop(0, _NCHUNK, body, 0)

    return k(idx3, table3)


def kernel(x, table):
    idx = jnp.reshape(x, (B_TOTAL, 1)).astype(jnp.int32)
    idx3 = jnp.reshape(idx * MR + jnp.arange(MR, dtype=jnp.int32),
                       (B_TOTAL * MR // 128, 128))
    table3 = jnp.reshape(
        jnp.pad(table, ((0, 0), (0, DIM_PAD - DIM))), (VOCAB * MR, 128)
    )
    out3 = _sc_gather(idx3, table3)
    out = jnp.reshape(out3, (B_TOTAL, DIM_PAD))[:, :DIM]
    return jnp.reshape(out, (x.shape[0], x.shape[1], DIM))


# speedup vs baseline: 1.4508x; 1.4376x over previous
import functools

import jax
import jax.numpy as jnp
from jax import lax
from jax.experimental import pallas as pl
from jax.experimental.pallas import tpu as pltpu
from jax.experimental.pallas import tpu_sc as plsc

VOCAB = 100000
DIM = 300
DIM_PAD = 384
B_TOTAL = 1024 * 200

_INFO = plsc.get_sparse_core_info()
_NC = _INFO.num_cores
_NS = _INFO.num_subcores
_NW = _NC * _NS
_B_PER_W = B_TOTAL // _NW
_CHUNK = 128
_NCHUNK = _B_PER_W // _CHUNK


def _sc_gather(idx3d, table_p):
    mesh = plsc.VectorSubcoreMesh(core_axis_name="c", subcore_axis_name="s")

    @functools.partial(
        pl.kernel,
        mesh=mesh,
        out_type=jax.ShapeDtypeStruct((B_TOTAL, DIM_PAD), jnp.float32),
        scratch_types=[
            pltpu.VMEM((_NCHUNK, 1, 128), jnp.int32),
            pltpu.VMEM((_CHUNK, DIM_PAD), jnp.float32),
            pltpu.SemaphoreType.DMA,
        ],
    )
    def k(idx_hbm, table_hbm, out_hbm, idx_v, rows_v, sem):
        wid = lax.axis_index("s") * _NC + lax.axis_index("c")
        base = wid * _B_PER_W
        pltpu.sync_copy(idx_hbm.at[pl.ds(wid * _NCHUNK, _NCHUNK)], idx_v)

        def body(c, carry):
            off = pl.multiple_of(c * _CHUNK, 8)
            pltpu.async_copy(
                table_hbm.at[idx_v.at[c, 0]], rows_v, sem
            ).wait()
            pltpu.sync_copy(rows_v, out_hbm.at[pl.ds(base + off, _CHUNK)])
            return carry

        lax.fori_loop(0, _NCHUNK, body, 0)

    return k(idx3d, table_p)


def kernel(x, table):
    idx3d = jnp.reshape(x, (_NW * _NCHUNK, 1, 128)).astype(jnp.int32)
    table_p = jnp.pad(table, ((0, 0), (0, DIM_PAD - DIM)))
    out_p = _sc_gather(idx3d, table_p)
    out = jnp.reshape(out_p, (B_TOTAL, DIM_PAD))[:, :DIM]
    return jnp.reshape(out, (x.shape[0], x.shape[1], DIM))


# speedup vs baseline: 1.4864x; 1.0245x over previous
import functools

import jax
import jax.numpy as jnp
from jax import lax
from jax.experimental import pallas as pl
from jax.experimental.pallas import tpu as pltpu
from jax.experimental.pallas import tpu_sc as plsc

VOCAB = 100000
DIM = 300
DIM_PAD = 384
B_TOTAL = 1024 * 200

_INFO = plsc.get_sparse_core_info()
_NC = _INFO.num_cores
_NS = _INFO.num_subcores
_NW = _NC * _NS
_B_PER_W = B_TOTAL // _NW
_CHUNK = 128
_NCHUNK = _B_PER_W // _CHUNK
_NPAIR = _NCHUNK // 2


def _sc_gather(idx3d, table_p):
    mesh = plsc.VectorSubcoreMesh(core_axis_name="c", subcore_axis_name="s")

    @functools.partial(
        pl.kernel,
        mesh=mesh,
        out_type=jax.ShapeDtypeStruct((B_TOTAL, DIM_PAD), jnp.float32),
        scratch_types=[
            pltpu.VMEM((_NCHUNK, 1, 128), jnp.int32),
            pltpu.VMEM((_CHUNK, DIM_PAD), jnp.float32),
            pltpu.VMEM((_CHUNK, DIM_PAD), jnp.float32),
            pltpu.SemaphoreType.DMA,
            pltpu.SemaphoreType.DMA,
        ],
    )
    def k(idx_hbm, table_hbm, out_hbm, idx_v, rows_a, rows_b, sem_a, sem_b):
        wid = lax.axis_index("s") * _NC + lax.axis_index("c")
        base = wid * _B_PER_W
        pltpu.sync_copy(idx_hbm.at[pl.ds(wid * _NCHUNK, _NCHUNK)], idx_v)

        def gather(c, buf, sem):
            return pltpu.async_copy(table_hbm.at[idx_v.at[c, 0]], buf, sem)

        def write(c, buf):
            off = pl.multiple_of(c * _CHUNK, 8)
            pltpu.sync_copy(buf, out_hbm.at[pl.ds(base + off, _CHUNK)])

        gather(0, rows_a, sem_a)

        def body(p, carry):
            c0 = p * 2
            gather(c0 + 1, rows_b, sem_b)
            pltpu.make_async_copy(
                table_hbm.at[idx_v.at[c0, 0]], rows_a, sem_a
            ).wait()
            write(c0, rows_a)

            @pl.when(p < _NPAIR - 1)
            def _():
                gather(c0 + 2, rows_a, sem_a)

            pltpu.make_async_copy(
                table_hbm.at[idx_v.at[c0 + 1, 0]], rows_b, sem_b
            ).wait()
            write(c0 + 1, rows_b)
            return carry

        lax.fori_loop(0, _NPAIR, body, 0)

    return k(idx3d, table_p)


def kernel(x, table):
    idx3d = jnp.reshape(x, (_NW * _NCHUNK, 1, 128)).astype(jnp.int32)
    table_p = jnp.pad(table, ((0, 0), (0, DIM_PAD - DIM)))
    out_p = _sc_gather(idx3d, table_p)
    out = jnp.reshape(out_p, (B_TOTAL, DIM_PAD))[:, :DIM]
    return jnp.reshape(out, (x.shape[0], x.shape[1], DIM))
